# baseline (device time: 37517 ns/iter reference)
import jax
import jax.numpy as jnp
from jax import lax
from jax.experimental import pallas as pl
from jax.experimental.pallas import tpu as pltpu

N_DEV = 4
N_LAYERS = 3
DELTA = (0, 3, 1, 2)
S1R_WAIT = (1, 3, 0, 2)
S1L_WAIT = (2, 0, 3, 1)
PART_PREV = (3, 2, 1, 0)


def kernel(x, Win0, Wout0, Win1, Wout1, Win2, Wout2):
    m_per, d = x.shape
    M = N_DEV * m_per

    def body(x_ref, win0_ref, wout0_ref, win1_ref, wout1_ref, win2_ref,
             wout2_ref, out_ref, psend, p_l, p_r, p_d, send_sems, recv_sems):
        my = lax.axis_index("i")
        left = jnp.mod(my - 1, N_DEV)
        right = jnp.mod(my + 1, N_DEV)
        diag = jnp.mod(my + 2, N_DEV)

        def mrc(src, dst, sem_idx, target):
            return pltpu.make_async_remote_copy(
                src_ref=src, dst_ref=dst,
                send_sem=send_sems.at[sem_idx],
                recv_sem=recv_sems.at[sem_idx],
                device_id=(target,),
                device_id_type=pl.DeviceIdType.MESH,
            )

        def rows(block):
            return pl.ds(block * m_per, m_per)

        barrier_sem = pltpu.get_barrier_semaphore()
        for nbr in (left, right):
            pl.semaphore_signal(
                barrier_sem, inc=1,
                device_id=(nbr,), device_id_type=pl.DeviceIdType.MESH,
            )
        pl.semaphore_wait(barrier_sem, 2)

        weights = ((win0_ref, wout0_ref), (win1_ref, wout1_ref),
                   (win2_ref, wout2_ref))

        def gemm(li, xc):
            win_ref, wout_ref = weights[li]
            hid = jnp.maximum(
                lax.dot(xc, win_ref[:, :],
                        preferred_element_type=jnp.float32), 0.0)
            return lax.dot(hid, wout_ref[:, :],
                           preferred_element_type=jnp.float32)

        def push_block(li, r, part, block):
            b = 3 + 12 * li + 3 * r
            ps = li % 2
            psend[ps, rows(block), :] = part.astype(jnp.bfloat16)
            s1l = mrc(psend.at[ps, rows(block), :],
                      p_r.at[li, rows(block), :], b + 0, left)
            s1r = mrc(psend.at[ps, rows(block), :],
                      p_l.at[li, rows(block), :], b + 1, right)
            s1d = mrc(psend.at[ps, rows(block), :],
                      p_d.at[li, rows(block), :], b + 2, diag)
            s1l.start()
            s1r.start()
            s1d.start()
            return s1l, s1r, s1d

        def finish_block(li_prev, part_prev, block):
            remote = (
                (p_l[li_prev, rows(block), :] + p_r[li_prev, rows(block), :])
                + p_d[li_prev, rows(block), :]
            )
            return part_prev + remote.astype(jnp.float32)

        out_ref[rows(my), :] = x_ref[:, :]
        g_l = mrc(x_ref, out_ref.at[rows(my), :], 0, left)
        g_r = mrc(x_ref, out_ref.at[rows(my), :], 1, right)
        g_d = mrc(x_ref, out_ref.at[rows(my), :], 2, diag)
        g_l.start()
        g_r.start()
        g_d.start()
        parts = [None] * N_DEV
        pend = [None] * N_DEV
        parts[0] = gemm(0, x_ref[:, :])
        pend[0] = push_block(0, 0, parts[0], jnp.mod(my + DELTA[0], N_DEV))
        g_r.wait()
        parts[1] = gemm(0, out_ref[rows(left), :])
        pend[1] = push_block(0, 1, parts[1], jnp.mod(my + DELTA[1], N_DEV))
        g_l.wait()
        parts[2] = gemm(0, out_ref[rows(right), :])
        pend[2] = push_block(0, 2, parts[2], jnp.mod(my + DELTA[2], N_DEV))
        g_d.wait()
        parts[3] = gemm(0, out_ref[rows(diag), :])
        pend[3] = push_block(0, 3, parts[3], jnp.mod(my + DELTA[3], N_DEV))

        for li in range(1, N_LAYERS):
            nparts = [None] * N_DEV
            npend = [None] * N_DEV
            for r in range(N_DEV):
                block = jnp.mod(my + 2 * li + DELTA[r], N_DEV)
                pend[S1R_WAIT[r]][1].wait()
                pend[S1L_WAIT[r]][0].wait()
                pend[r][2].wait()
                xc = finish_block(li - 1, parts[PART_PREV[r]], block)
                nparts[r] = gemm(li, xc)
                npend[r] = push_block(li, r, nparts[r], block)
            parts, pend = nparts, npend

        for r in range(N_DEV):
            block = jnp.mod(my + 2 * N_LAYERS + DELTA[r], N_DEV)
            pend[S1R_WAIT[r]][1].wait()
            pend[S1L_WAIT[r]][0].wait()
            pend[r][2].wait()
            out_ref[rows(block), :] = finish_block(
                N_LAYERS - 1, parts[PART_PREV[r]], block)

    n_sems = 3 + N_LAYERS * N_DEV * 3
    return pl.pallas_call(
        body,
        out_shape=jax.ShapeDtypeStruct((M, d), jnp.float32),
        in_specs=[pl.BlockSpec(memory_space=pltpu.VMEM)] * 7,
        out_specs=pl.BlockSpec(memory_space=pltpu.VMEM),
        scratch_shapes=[
            pltpu.VMEM((2, M, d), jnp.bfloat16),
            pltpu.VMEM((N_LAYERS, M, d), jnp.bfloat16),
            pltpu.VMEM((N_LAYERS, M, d), jnp.bfloat16),
            pltpu.VMEM((N_LAYERS, M, d), jnp.bfloat16),
            pltpu.SemaphoreType.DMA((n_sems,)),
            pltpu.SemaphoreType.DMA((n_sems,)),
        ],
        compiler_params=pltpu.CompilerParams(collective_id=0),
    )(x, Win0, Wout0, Win1, Wout1, Win2, Wout2)


# device time: 37409 ns/iter; 1.0029x vs baseline; 1.0029x over previous
import jax
import jax.numpy as jnp
from jax import lax
from jax.experimental import pallas as pl
from jax.experimental.pallas import tpu as pltpu

N_DEV = 4
N_LAYERS = 3
DELTA = (0, 3, 1, 2)
S1R_WAIT = (1, 3, 0, 2)
S1L_WAIT = (2, 0, 3, 1)
PART_PREV = (3, 2, 1, 0)


def kernel(x, Win0, Wout0, Win1, Wout1, Win2, Wout2):
    m_per, d = x.shape
    M = N_DEV * m_per

    def body(x_ref, win0_ref, wout0_ref, win1_ref, wout1_ref, win2_ref,
             wout2_ref, out_ref, psend, p_l, p_r, p_d, send_sems, recv_sems):
        my = lax.axis_index("i")
        left = jnp.mod(my - 1, N_DEV)
        right = jnp.mod(my + 1, N_DEV)
        diag = jnp.mod(my + 2, N_DEV)

        def mrc(src, dst, sem_idx, target):
            return pltpu.make_async_remote_copy(
                src_ref=src, dst_ref=dst,
                send_sem=send_sems.at[sem_idx],
                recv_sem=recv_sems.at[sem_idx],
                device_id=(target,),
                device_id_type=pl.DeviceIdType.MESH,
            )

        def rows(block):
            return pl.ds(block * m_per, m_per)

        barrier_sem = pltpu.get_barrier_semaphore()
        for nbr in (left, right):
            pl.semaphore_signal(
                barrier_sem, inc=1,
                device_id=(nbr,), device_id_type=pl.DeviceIdType.MESH,
            )

        weights = ((win0_ref, wout0_ref), (win1_ref, wout1_ref),
                   (win2_ref, wout2_ref))

        def gemm(li, xc):
            win_ref, wout_ref = weights[li]
            hid = jnp.maximum(
                lax.dot(xc, win_ref[:, :],
                        preferred_element_type=jnp.float32), 0.0)
            return lax.dot(hid, wout_ref[:, :],
                           preferred_element_type=jnp.float32)

        def push_block(li, r, part, block):
            b = 3 + 12 * li + 3 * r
            ps = li % 2
            psend[ps, rows(block), :] = part.astype(jnp.bfloat16)
            s1l = mrc(psend.at[ps, rows(block), :],
                      p_r.at[li, rows(block), :], b + 0, left)
            s1r = mrc(psend.at[ps, rows(block), :],
                      p_l.at[li, rows(block), :], b + 1, right)
            s1d = mrc(psend.at[ps, rows(block), :],
                      p_d.at[li, rows(block), :], b + 2, diag)
            s1l.start()
            s1r.start()
            s1d.start()
            return s1l, s1r, s1d

        def finish_block(li_prev, part_prev, block):
            remote = (
                (p_l[li_prev, rows(block), :] + p_r[li_prev, rows(block), :])
                + p_d[li_prev, rows(block), :]
            )
            return part_prev + remote.astype(jnp.float32)

        out_ref[rows(my), :] = x_ref[:, :]
        parts = [None] * N_DEV
        pend = [None] * N_DEV
        parts[0] = gemm(0, x_ref[:, :])
        pl.semaphore_wait(barrier_sem, 2)
        g_l = mrc(x_ref, out_ref.at[rows(my), :], 0, left)
        g_r = mrc(x_ref, out_ref.at[rows(my), :], 1, right)
        g_d = mrc(x_ref, out_ref.at[rows(my), :], 2, diag)
        g_l.start()
        g_r.start()
        g_d.start()
        pend[0] = push_block(0, 0, parts[0], jnp.mod(my + DELTA[0], N_DEV))
        g_r.wait()
        parts[1] = gemm(0, out_ref[rows(left), :])
        pend[1] = push_block(0, 1, parts[1], jnp.mod(my + DELTA[1], N_DEV))
        g_l.wait()
        parts[2] = gemm(0, out_ref[rows(right), :])
        pend[2] = push_block(0, 2, parts[2], jnp.mod(my + DELTA[2], N_DEV))
        g_d.wait()
        parts[3] = gemm(0, out_ref[rows(diag), :])
        pend[3] = push_block(0, 3, parts[3], jnp.mod(my + DELTA[3], N_DEV))

        for li in range(1, N_LAYERS):
            nparts = [None] * N_DEV
            npend = [None] * N_DEV
            for r in range(N_DEV):
                block = jnp.mod(my + 2 * li + DELTA[r], N_DEV)
                pend[S1R_WAIT[r]][1].wait()
                pend[S1L_WAIT[r]][0].wait()
                pend[r][2].wait()
                xc = finish_block(li - 1, parts[PART_PREV[r]], block)
                nparts[r] = gemm(li, xc)
                npend[r] = push_block(li, r, nparts[r], block)
            parts, pend = nparts, npend

        for r in range(N_DEV):
            block = jnp.mod(my + 2 * N_LAYERS + DELTA[r], N_DEV)
            pend[S1R_WAIT[r]][1].wait()
            pend[S1L_WAIT[r]][0].wait()
            pend[r][2].wait()
            out_ref[rows(block), :] = finish_block(
                N_LAYERS - 1, parts[PART_PREV[r]], block)

    n_sems = 3 + N_LAYERS * N_DEV * 3
    return pl.pallas_call(
        body,
        out_shape=jax.ShapeDtypeStruct((M, d), jnp.float32),
        in_specs=[pl.BlockSpec(memory_space=pltpu.VMEM)] * 7,
        out_specs=pl.BlockSpec(memory_space=pltpu.VMEM),
        scratch_shapes=[
            pltpu.VMEM((2, M, d), jnp.bfloat16),
            pltpu.VMEM((N_LAYERS, M, d), jnp.bfloat16),
            pltpu.VMEM((N_LAYERS, M, d), jnp.bfloat16),
            pltpu.VMEM((N_LAYERS, M, d), jnp.bfloat16),
            pltpu.SemaphoreType.DMA((n_sems,)),
            pltpu.SemaphoreType.DMA((n_sems,)),
        ],
        compiler_params=pltpu.CompilerParams(collective_id=0),
    )(x, Win0, Wout0, Win1, Wout1, Win2, Wout2)
